# Initial kernel scaffold; baseline (speedup 1.0000x reference)
import jax
import jax.numpy as jnp
from jax.experimental import pallas as pl


def kernel(x, emb_weight, fc_weight, fc_bias):
    raise NotImplementedError("write your pallas kernel here")


# speedup vs baseline: 39.0180x; 39.0180x over previous
import functools

import jax
import jax.numpy as jnp
from jax import lax
from jax.experimental import pallas as pl
from jax.experimental.pallas import tpu as pltpu
from jax.experimental.pallas import tpu_sc as plsc

VOCAB = 1000
EMB = 128
B = 16384
L = 200

LANES = 16
VP = 1008
NC, NS = 2, 16
NW = NC * NS
ROWS_PER_W = B // NW
CHUNK = 64
NCHUNK = ROWS_PER_W // CHUNK
NGROUPS = (L + LANES - 1) // LANES
TAIL = L - (NGROUPS - 1) * LANES


def _hist_body(x_hbm, counts_hbm, xv, cv):
    c = lax.axis_index("c")
    s = lax.axis_index("s")
    wid = s * NC + c
    base_row = wid * ROWS_PER_W

    zeros16 = jnp.zeros((LANES,), jnp.float32)
    ones16 = jnp.ones((LANES,), jnp.float32)
    tail_mask = lax.iota(jnp.int32, LANES) < TAIL

    for chunk in range(NCHUNK):
        row0 = base_row + chunk * CHUNK

        pltpu.sync_copy(
            x_hbm.at[pl.ds(row0 * L, CHUNK * L)],
            xv.at[pl.ds(0, CHUNK * L)],
        )

        def zero_body(i, _):
            cv[pl.ds(i * LANES, LANES)] = zeros16
            return 0

        lax.fori_loop(0, CHUNK * VP // LANES, zero_body, 0)

        def row_body(r, _):
            rbase = r * VP
            xbase = r * L
            for g in range(NGROUPS):
                toks = xv[pl.ds(xbase + g * LANES, LANES)]
                flat = toks + rbase
                if g == NGROUPS - 1:
                    plsc.addupdate_scatter(cv, [flat], ones16, mask=tail_mask)
                else:
                    plsc.addupdate_scatter(cv, [flat], ones16)
            return 0

        lax.fori_loop(0, CHUNK, row_body, 0)

        pltpu.sync_copy(
            cv,
            counts_hbm.at[pl.ds(row0 * VP, CHUNK * VP)],
        )


_hist_call = functools.partial(
    pl.kernel,
    out_type=jax.ShapeDtypeStruct((B * VP,), jnp.float32),
    mesh=plsc.VectorSubcoreMesh(core_axis_name="c", subcore_axis_name="s"),
    compiler_params=pltpu.CompilerParams(needs_layout_passes=False),
    scratch_types=[
        pltpu.VMEM((CHUNK * L + LANES,), jnp.int32),
        pltpu.VMEM((CHUNK * VP,), jnp.float32),
    ],
)(_hist_body)


def _m2_body(emb_ref, fc_ref, out_ref):
    out_ref[...] = lax.dot_general(
        emb_ref[...], fc_ref[...],
        (((1,), (1,)), ((), ())),
        preferred_element_type=jnp.float32,
    ) * (1.0 / L)


def _mm_body(cnt_ref, m2_ref, b_ref, out_ref):
    out_ref[...] = (
        jnp.dot(cnt_ref[...], m2_ref[...], preferred_element_type=jnp.float32)
        + b_ref[...]
    )


BB = 1024


def kernel(x, emb_weight, fc_weight, fc_bias):
    x_flat = x.reshape(-1)
    emb_pad = jnp.pad(emb_weight, ((0, VP - VOCAB), (0, 0)))

    m2 = pl.pallas_call(
        _m2_body,
        out_shape=jax.ShapeDtypeStruct((VP, EMB), jnp.float32),
    )(emb_pad, fc_weight)

    counts = _hist_call(x_flat).reshape(B, VP)

    out = pl.pallas_call(
        _mm_body,
        grid=(B // BB,),
        in_specs=[
            pl.BlockSpec((BB, VP), lambda i: (i, 0)),
            pl.BlockSpec((VP, EMB), lambda i: (0, 0)),
            pl.BlockSpec((1, EMB), lambda i: (0, 0)),
        ],
        out_specs=pl.BlockSpec((BB, EMB), lambda i: (i, 0)),
        out_shape=jax.ShapeDtypeStruct((B, EMB), jnp.float32),
    )(counts, m2, fc_bias.reshape(1, EMB))

    return out


# speedup vs baseline: 54.0173x; 1.3844x over previous
import functools

import jax
import jax.numpy as jnp
from jax import lax
from jax.experimental import pallas as pl
from jax.experimental.pallas import tpu as pltpu
from jax.experimental.pallas import tpu_sc as plsc

VOCAB = 1000
EMB = 128
B = 16384
L = 200

LANES = 16
VP = 1008
NC, NS = 2, 16
NW = NC * NS
ROWS_PER_W = B // NW
CHUNK = 32
NCHUNK = ROWS_PER_W // CHUNK
NBUF = 2
ZUNROLL = 16
NGROUPS = (L + LANES - 1) // LANES
TAIL = L - (NGROUPS - 1) * LANES


def _hist_body(x_hbm, counts_hbm, xv0, xv1, cv0, cv1, sem0, sem1):
    c = lax.axis_index("c")
    s = lax.axis_index("s")
    wid = s * NC + c
    base_row = wid * ROWS_PER_W
    xv = [xv0, xv1]
    cv = [cv0, cv1]
    sems = [sem0, sem1]

    zeros16 = jnp.zeros((LANES,), jnp.float32)
    ones16 = jnp.ones((LANES,), jnp.float32)
    tail_mask = lax.iota(jnp.int32, LANES) < TAIL

    def zero_body(i, _):
        for p in range(NBUF):
            for k in range(ZUNROLL):
                cv[p][pl.ds(i * (LANES * ZUNROLL) + k * LANES, LANES)] = zeros16
        return 0

    lax.fori_loop(0, CHUNK * VP // (LANES * ZUNROLL), zero_body, 0)

    def scatter_rows(p, value, mask_only_tail):
        def row_body(r, _):
            rbase = r * VP
            xbase = r * L
            for g in range(NGROUPS):
                toks = xv[p][pl.ds(xbase + g * LANES, LANES)]
                flat = toks + rbase
                msk = tail_mask if g == NGROUPS - 1 else None
                if value is None:
                    if msk is None:
                        plsc.store_scatter(cv[p], [flat], zeros16)
                    else:
                        plsc.store_scatter(cv[p], [flat], zeros16, mask=msk)
                else:
                    if msk is None:
                        plsc.addupdate_scatter(cv[p], [flat], ones16)
                    else:
                        plsc.addupdate_scatter(cv[p], [flat], ones16, mask=msk)
            return 0

        lax.fori_loop(0, CHUNK, row_body, 0)

    out_dma = [None] * NBUF
    for chunk in range(NCHUNK):
        p = chunk % NBUF
        row0 = base_row + chunk * CHUNK

        if out_dma[p] is not None:
            out_dma[p].wait()
            scatter_rows(p, None, None)

        pltpu.sync_copy(
            x_hbm.at[pl.ds(row0 * L, CHUNK * L)],
            xv[p].at[pl.ds(0, CHUNK * L)],
        )

        scatter_rows(p, 1.0, None)

        out_dma[p] = pltpu.async_copy(
            cv[p],
            counts_hbm.at[pl.ds(row0 * VP, CHUNK * VP)],
            sems[p],
        )

    for p in range(NBUF):
        if out_dma[p] is not None:
            out_dma[p].wait()


_hist_call = functools.partial(
    pl.kernel,
    out_type=jax.ShapeDtypeStruct((B * VP,), jnp.float32),
    mesh=plsc.VectorSubcoreMesh(core_axis_name="c", subcore_axis_name="s"),
    compiler_params=pltpu.CompilerParams(needs_layout_passes=False),
    scratch_types=[
        pltpu.VMEM((CHUNK * L + LANES,), jnp.int32),
        pltpu.VMEM((CHUNK * L + LANES,), jnp.int32),
        pltpu.VMEM((CHUNK * VP,), jnp.float32),
        pltpu.VMEM((CHUNK * VP,), jnp.float32),
        pltpu.SemaphoreType.DMA,
        pltpu.SemaphoreType.DMA,
    ],
)(_hist_body)


def _m2_body(emb_ref, fc_ref, out_ref):
    out_ref[...] = lax.dot_general(
        emb_ref[...], fc_ref[...],
        (((1,), (1,)), ((), ())),
        preferred_element_type=jnp.float32,
    ) * (1.0 / L)


def _mm_body(cnt_ref, m2_ref, b_ref, out_ref):
    out_ref[...] = (
        jnp.dot(
            cnt_ref[...].astype(jnp.bfloat16),
            m2_ref[...].astype(jnp.bfloat16),
            preferred_element_type=jnp.float32,
        )
        + b_ref[...]
    )


BB = 1024


def kernel(x, emb_weight, fc_weight, fc_bias):
    x_flat = x.reshape(-1)
    emb_pad = jnp.pad(emb_weight, ((0, VP - VOCAB), (0, 0)))

    m2 = pl.pallas_call(
        _m2_body,
        out_shape=jax.ShapeDtypeStruct((VP, EMB), jnp.float32),
    )(emb_pad, fc_weight)

    counts = _hist_call(x_flat).reshape(B, VP)

    out = pl.pallas_call(
        _mm_body,
        grid=(B // BB,),
        in_specs=[
            pl.BlockSpec((BB, VP), lambda i: (i, 0)),
            pl.BlockSpec((VP, EMB), lambda i: (0, 0)),
            pl.BlockSpec((1, EMB), lambda i: (0, 0)),
        ],
        out_specs=pl.BlockSpec((BB, EMB), lambda i: (i, 0)),
        out_shape=jax.ShapeDtypeStruct((B, EMB), jnp.float32),
    )(counts, m2, fc_bias.reshape(1, EMB))

    return out


# speedup vs baseline: 64.5590x; 1.1952x over previous
import functools

import jax
import jax.numpy as jnp
from jax import lax
from jax.experimental import pallas as pl
from jax.experimental.pallas import tpu as pltpu
from jax.experimental.pallas import tpu_sc as plsc

VOCAB = 1000
EMB = 128
B = 16384
L = 200

LANES = 16
VP = 1008
NC, NS = 2, 16
NW = NC * NS
NSLICE = 4
BSL = B // NSLICE
ROWS_PER_W = BSL // NW
CHUNK = 32
NCHUNK = ROWS_PER_W // CHUNK
NBUF = 2
ZUNROLL = 16
NGROUPS = (L + LANES - 1) // LANES
TAIL = L - (NGROUPS - 1) * LANES


def _hist_body(x_hbm, counts_hbm, xv0, xv1, cv0, cv1, sem0, sem1, *, slice_idx):
    c = lax.axis_index("c")
    s = lax.axis_index("s")
    wid = s * NC + c
    base_row = wid * ROWS_PER_W
    gbase_row = slice_idx * BSL + base_row
    xv = [xv0, xv1]
    cv = [cv0, cv1]
    sems = [sem0, sem1]

    zeros16 = jnp.zeros((LANES,), jnp.float32)
    ones16 = jnp.ones((LANES,), jnp.float32)
    tail_mask = lax.iota(jnp.int32, LANES) < TAIL

    def zero_body(i, _):
        for p in range(NBUF):
            for k in range(ZUNROLL):
                cv[p][pl.ds(i * (LANES * ZUNROLL) + k * LANES, LANES)] = zeros16
        return 0

    lax.fori_loop(0, CHUNK * VP // (LANES * ZUNROLL), zero_body, 0)

    def scatter_rows(p, value, mask_only_tail):
        def row_body(r, _):
            rbase = r * VP
            xbase = r * L
            for g in range(NGROUPS):
                toks = xv[p][pl.ds(xbase + g * LANES, LANES)]
                flat = toks + rbase
                msk = tail_mask if g == NGROUPS - 1 else None
                if value is None:
                    if msk is None:
                        plsc.store_scatter(cv[p], [flat], zeros16)
                    else:
                        plsc.store_scatter(cv[p], [flat], zeros16, mask=msk)
                else:
                    if msk is None:
                        plsc.addupdate_scatter(cv[p], [flat], ones16)
                    else:
                        plsc.addupdate_scatter(cv[p], [flat], ones16, mask=msk)
            return 0

        lax.fori_loop(0, CHUNK, row_body, 0)

    out_dma = [None] * NBUF
    for chunk in range(NCHUNK):
        p = chunk % NBUF
        row0 = base_row + chunk * CHUNK
        grow0 = gbase_row + chunk * CHUNK

        if out_dma[p] is not None:
            out_dma[p].wait()
            scatter_rows(p, None, None)

        pltpu.sync_copy(
            x_hbm.at[pl.ds(grow0 * L, CHUNK * L)],
            xv[p].at[pl.ds(0, CHUNK * L)],
        )

        scatter_rows(p, 1.0, None)

        out_dma[p] = pltpu.async_copy(
            cv[p],
            counts_hbm.at[pl.ds(row0 * VP, CHUNK * VP)],
            sems[p],
        )

    for p in range(NBUF):
        if out_dma[p] is not None:
            out_dma[p].wait()


def _make_hist_call(slice_idx):
    return functools.partial(
        pl.kernel,
        out_type=jax.ShapeDtypeStruct((BSL * VP,), jnp.float32),
        mesh=plsc.VectorSubcoreMesh(core_axis_name="c", subcore_axis_name="s"),
        compiler_params=pltpu.CompilerParams(needs_layout_passes=False),
        scratch_types=[
            pltpu.VMEM((CHUNK * L + LANES,), jnp.int32),
            pltpu.VMEM((CHUNK * L + LANES,), jnp.int32),
            pltpu.VMEM((CHUNK * VP,), jnp.float32),
            pltpu.VMEM((CHUNK * VP,), jnp.float32),
            pltpu.SemaphoreType.DMA,
            pltpu.SemaphoreType.DMA,
        ],
        name=f"hist_slice{slice_idx}",
    )(functools.partial(_hist_body, slice_idx=slice_idx))


_hist_calls = [_make_hist_call(i) for i in range(NSLICE)]


def _m2_body(emb_ref, fc_ref, out_ref):
    out_ref[...] = lax.dot_general(
        emb_ref[...], fc_ref[...],
        (((1,), (1,)), ((), ())),
        preferred_element_type=jnp.float32,
    ) * (1.0 / L)


def _mm_body(cnt_ref, m2_ref, b_ref, out_ref):
    out_ref[...] = (
        jnp.dot(
            cnt_ref[...].astype(jnp.bfloat16),
            m2_ref[...].astype(jnp.bfloat16),
            preferred_element_type=jnp.float32,
        )
        + b_ref[...]
    )


BB = 1024


def kernel(x, emb_weight, fc_weight, fc_bias):
    x_flat = x.reshape(-1)
    emb_pad = jnp.pad(emb_weight, ((0, VP - VOCAB), (0, 0)))

    m2 = pl.pallas_call(
        _m2_body,
        out_shape=jax.ShapeDtypeStruct((VP, EMB), jnp.float32),
    )(emb_pad, fc_weight)

    bias2d = fc_bias.reshape(1, EMB)
    outs = []
    for i in range(NSLICE):
        counts = _hist_calls[i](x_flat).reshape(BSL, VP)
        outs.append(
            pl.pallas_call(
                _mm_body,
                grid=(BSL // BB,),
                in_specs=[
                    pl.BlockSpec((BB, VP), lambda i: (i, 0)),
                    pl.BlockSpec((VP, EMB), lambda i: (0, 0)),
                    pl.BlockSpec((1, EMB), lambda i: (0, 0)),
                ],
                out_specs=pl.BlockSpec((BB, EMB), lambda i: (i, 0)),
                out_shape=jax.ShapeDtypeStruct((BSL, EMB), jnp.float32),
            )(counts, m2, bias2d)
        )

    return jnp.concatenate(outs, axis=0)


# speedup vs baseline: 74.2528x; 1.1502x over previous
import functools

import jax
import jax.numpy as jnp
from jax import lax
from jax.experimental import pallas as pl
from jax.experimental.pallas import tpu as pltpu
from jax.experimental.pallas import tpu_sc as plsc

VOCAB = 1000
EMB = 128
B = 16384
L = 200

LANES = 16
VP = 1024
NC, NS = 2, 16
NW = NC * NS
NSLICE = 4
BSL = B // NSLICE
ROWS_PER_W = BSL // NW
CHUNK = 32
NCHUNK = ROWS_PER_W // CHUNK
NBUF = 2
ZUNROLL = 16
NFULL = L // LANES
TAIL_OFF = L - LANES


def _hist_body(x_hbm, counts_hbm, xv0, xv1, cv0, cv1, sem0, sem1, *, slice_idx):
    c = lax.axis_index("c")
    s = lax.axis_index("s")
    wid = s * NC + c
    base_row = wid * ROWS_PER_W
    gbase_row = slice_idx * BSL + base_row
    xv = [xv0, xv1]
    cv = [cv0, cv1]
    sems = [sem0, sem1]

    zeros16 = jnp.zeros((LANES,), jnp.float32)
    ones16 = jnp.ones((LANES,), jnp.float32)
    tail_mask = lax.iota(jnp.int32, LANES) >= (LANES - (L - NFULL * LANES))

    def zero_body(i, _):
        for p in range(NBUF):
            for k in range(ZUNROLL):
                cv[p][pl.ds(i * (LANES * ZUNROLL) + k * LANES, LANES)] = zeros16
        return 0

    lax.fori_loop(0, CHUNK * VP // (LANES * ZUNROLL), zero_body, 0)

    def scatter_rows(p, value):
        offs = [g * LANES for g in range(NFULL)] + [TAIL_OFF]

        def row_body(r, _):
            rbase = r * VP
            for i, off in enumerate(offs):
                toks = xv[p][r, pl.ds(off, LANES)]
                flat = toks + rbase
                msk = tail_mask if i == NFULL else None
                if value is None:
                    if msk is None:
                        plsc.store_scatter(cv[p], [flat], zeros16)
                    else:
                        plsc.store_scatter(cv[p], [flat], zeros16, mask=msk)
                else:
                    if msk is None:
                        plsc.addupdate_scatter(cv[p], [flat], ones16)
                    else:
                        plsc.addupdate_scatter(cv[p], [flat], ones16, mask=msk)
            return 0

        lax.fori_loop(0, CHUNK, row_body, 0)

    out_dma = [None] * NBUF
    for chunk in range(NCHUNK):
        p = chunk % NBUF
        row0 = base_row + chunk * CHUNK
        grow0 = gbase_row + chunk * CHUNK

        if out_dma[p] is not None:
            out_dma[p].wait()
            scatter_rows(p, None)

        pltpu.sync_copy(x_hbm.at[pl.ds(grow0, CHUNK)], xv[p])

        scatter_rows(p, 1.0)

        out_dma[p] = pltpu.async_copy(
            cv[p],
            counts_hbm.at[pl.ds(row0 * VP, CHUNK * VP)],
            sems[p],
        )

    for p in range(NBUF):
        if out_dma[p] is not None:
            out_dma[p].wait()


def _make_hist_call(slice_idx):
    return functools.partial(
        pl.kernel,
        out_type=jax.ShapeDtypeStruct((BSL * VP,), jnp.float32),
        mesh=plsc.VectorSubcoreMesh(core_axis_name="c", subcore_axis_name="s"),
        compiler_params=pltpu.CompilerParams(needs_layout_passes=False),
        scratch_types=[
            pltpu.VMEM((CHUNK, L), jnp.int32),
            pltpu.VMEM((CHUNK, L), jnp.int32),
            pltpu.VMEM((CHUNK * VP,), jnp.float32),
            pltpu.VMEM((CHUNK * VP,), jnp.float32),
            pltpu.SemaphoreType.DMA,
            pltpu.SemaphoreType.DMA,
        ],
        name=f"hist_slice{slice_idx}",
    )(functools.partial(_hist_body, slice_idx=slice_idx))


_hist_calls = [_make_hist_call(i) for i in range(NSLICE)]


def _m2_body(emb_ref, fc_ref, out_ref):
    out_ref[...] = lax.dot_general(
        emb_ref[...], fc_ref[...],
        (((1,), (1,)), ((), ())),
        preferred_element_type=jnp.float32,
    ) * (1.0 / L)


def _mm_body(cnt_ref, m2_ref, b_ref, out_ref):
    out_ref[...] = (
        jnp.dot(
            cnt_ref[...].astype(jnp.bfloat16),
            m2_ref[...].astype(jnp.bfloat16),
            preferred_element_type=jnp.float32,
        )
        + b_ref[...]
    )


BB = 1024


def kernel(x, emb_weight, fc_weight, fc_bias):
    emb_pad = jnp.pad(emb_weight, ((0, VP - VOCAB), (0, 0)))

    m2 = pl.pallas_call(
        _m2_body,
        out_shape=jax.ShapeDtypeStruct((VP, EMB), jnp.float32),
    )(emb_pad, fc_weight)

    bias2d = fc_bias.reshape(1, EMB)
    outs = []
    for i in range(NSLICE):
        counts = _hist_calls[i](x).reshape(BSL, VP)
        outs.append(
            pl.pallas_call(
                _mm_body,
                grid=(BSL // BB,),
                in_specs=[
                    pl.BlockSpec((BB, VP), lambda i: (i, 0)),
                    pl.BlockSpec((VP, EMB), lambda i: (0, 0)),
                    pl.BlockSpec((1, EMB), lambda i: (0, 0)),
                ],
                out_specs=pl.BlockSpec((BB, EMB), lambda i: (i, 0)),
                out_shape=jax.ShapeDtypeStruct((BSL, EMB), jnp.float32),
            )(counts, m2, bias2d)
        )

    return jnp.concatenate(outs, axis=0)


# speedup vs baseline: 75.7052x; 1.0196x over previous
import functools

import jax
import jax.numpy as jnp
from jax import lax
from jax.experimental import pallas as pl
from jax.experimental.pallas import tpu as pltpu
from jax.experimental.pallas import tpu_sc as plsc

VOCAB = 1000
EMB = 128
B = 16384
L = 200

LANES = 16
VP = 1024
NC, NS = 2, 16
NW = NC * NS
NSLICE = 4
BSL = B // NSLICE
ROWS_PER_W = BSL // NW
CHUNK = 32
NCHUNK = ROWS_PER_W // CHUNK
NBUF = 2
ZUNROLL = 16
NFULL = L // LANES
TAIL_OFF = L - LANES


def _hist_body(x_hbm, counts_hbm, xv0, xv1, cv0, cv1, sem0, sem1, *, slice_idx):
    c = lax.axis_index("c")
    s = lax.axis_index("s")
    wid = s * NC + c
    base_row = wid * ROWS_PER_W
    gbase_row = slice_idx * BSL + base_row
    xv = [xv0, xv1]
    cv = [cv0, cv1]
    sems = [sem0, sem1]

    zeros16 = jnp.zeros((LANES,), jnp.float32)
    ones16 = jnp.ones((LANES,), jnp.float32)
    tail_mask = lax.iota(jnp.int32, LANES) >= (LANES - (L - NFULL * LANES))

    def zero_body(r, _):
        for p in range(NBUF):
            for k in range(VP // LANES):
                cv[p][r, pl.ds(k * LANES, LANES)] = zeros16
        return 0

    lax.fori_loop(0, CHUNK, zero_body, 0)

    def scatter_rows(p, value):
        offs = [g * LANES for g in range(NFULL)] + [TAIL_OFF]

        def row_body(r, _):
            rows16 = jnp.full((LANES,), 0, jnp.int32) + r
            for i, off in enumerate(offs):
                toks = xv[p][r, pl.ds(off, LANES)]
                msk = tail_mask if i == NFULL else None
                if value is None:
                    if msk is None:
                        plsc.store_scatter(cv[p], [rows16, toks], zeros16)
                    else:
                        plsc.store_scatter(cv[p], [rows16, toks], zeros16, mask=msk)
                else:
                    if msk is None:
                        plsc.addupdate_scatter(cv[p], [rows16, toks], ones16)
                    else:
                        plsc.addupdate_scatter(cv[p], [rows16, toks], ones16, mask=msk)
            return 0

        lax.fori_loop(0, CHUNK, row_body, 0)

    out_dma = [None] * NBUF
    for chunk in range(NCHUNK):
        p = chunk % NBUF
        row0 = base_row + chunk * CHUNK
        grow0 = gbase_row + chunk * CHUNK

        if out_dma[p] is not None:
            out_dma[p].wait()
            scatter_rows(p, None)

        pltpu.sync_copy(x_hbm.at[pl.ds(grow0, CHUNK)], xv[p])

        scatter_rows(p, 1.0)

        out_dma[p] = pltpu.async_copy(
            cv[p],
            counts_hbm.at[pl.ds(row0, CHUNK)],
            sems[p],
        )

    for p in range(NBUF):
        if out_dma[p] is not None:
            out_dma[p].wait()


def _make_hist_call(slice_idx):
    return functools.partial(
        pl.kernel,
        out_type=jax.ShapeDtypeStruct((BSL, VP), jnp.float32),
        mesh=plsc.VectorSubcoreMesh(core_axis_name="c", subcore_axis_name="s"),
        compiler_params=pltpu.CompilerParams(needs_layout_passes=False),
        scratch_types=[
            pltpu.VMEM((CHUNK, L), jnp.int32),
            pltpu.VMEM((CHUNK, L), jnp.int32),
            pltpu.VMEM((CHUNK, VP), jnp.float32),
            pltpu.VMEM((CHUNK, VP), jnp.float32),
            pltpu.SemaphoreType.DMA,
            pltpu.SemaphoreType.DMA,
        ],
        name=f"hist_slice{slice_idx}",
    )(functools.partial(_hist_body, slice_idx=slice_idx))


_hist_calls = [_make_hist_call(i) for i in range(NSLICE)]


def _m2_body(emb_ref, fc_ref, out_ref):
    out_ref[...] = lax.dot_general(
        emb_ref[...], fc_ref[...],
        (((1,), (1,)), ((), ())),
        preferred_element_type=jnp.float32,
    ) * (1.0 / L)


def _mm_body(cnt_ref, m2_ref, b_ref, acc_ref, out_ref):
    del acc_ref
    out_ref[...] = (
        jnp.dot(
            cnt_ref[...].astype(jnp.bfloat16),
            m2_ref[...].astype(jnp.bfloat16),
            preferred_element_type=jnp.float32,
        )
        + b_ref[...]
    )


BB = 1024


def kernel(x, emb_weight, fc_weight, fc_bias):
    emb_pad = jnp.pad(emb_weight, ((0, VP - VOCAB), (0, 0)))

    m2 = pl.pallas_call(
        _m2_body,
        out_shape=jax.ShapeDtypeStruct((VP, EMB), jnp.float32),
    )(emb_pad, fc_weight)

    bias2d = fc_bias.reshape(1, EMB)
    out = jnp.zeros((B, EMB), jnp.float32)
    for i in range(NSLICE):
        counts = _hist_calls[i](x)
        base_blk = i * (BSL // BB)
        out = pl.pallas_call(
            _mm_body,
            grid=(BSL // BB,),
            in_specs=[
                pl.BlockSpec((BB, VP), lambda j: (j, 0)),
                pl.BlockSpec((VP, EMB), lambda j: (0, 0)),
                pl.BlockSpec((1, EMB), lambda j: (0, 0)),
                pl.BlockSpec((BB, EMB), lambda j, b=base_blk: (b + j, 0)),
            ],
            out_specs=pl.BlockSpec((BB, EMB), lambda j, b=base_blk: (b + j, 0)),
            out_shape=jax.ShapeDtypeStruct((B, EMB), jnp.float32),
            input_output_aliases={3: 0},
        )(counts, m2, bias2d, out)

    return out
